# Initial kernel scaffold; baseline (speedup 1.0000x reference)
import jax
import jax.numpy as jnp
from jax.experimental import pallas as pl


def kernel(x, table, W1, b1, W2, b2):
    raise NotImplementedError("write your pallas kernel here")


# speedup vs baseline: 2.4997x; 2.4997x over previous
import functools

import jax
import jax.numpy as jnp
from jax import lax
from jax.experimental import pallas as pl
from jax.experimental.pallas import tpu as pltpu
from jax.experimental.pallas import tpu_sc as plsc

B = 16384
HIST = 50
D = 64
H = 256
C = 2

NC = 2
NS = 16
NW = NC * NS
ROWS_W = B // NW
RPC = 2
IPC = RPC * HIST
NCHUNK = ROWS_W // RPC
LANES = 16
NV = D // LANES


def _sc_pool(x_r, table):
    mesh = plsc.VectorSubcoreMesh(core_axis_name="c", subcore_axis_name="s")

    @functools.partial(
        pl.kernel,
        mesh=mesh,
        compiler_params=pltpu.CompilerParams(use_tc_tiling_on_sc=False),
        out_type=jax.ShapeDtypeStruct((B, D), jnp.float32),
        scratch_types=[
            pltpu.VMEM((NCHUNK, IPC), jnp.int32),
            pltpu.VMEM((2, IPC, D), jnp.float32),
            pltpu.VMEM((ROWS_W, D), jnp.float32),
            pltpu.SemaphoreType.DMA,
            pltpu.SemaphoreType.DMA,
        ],
    )
    def k(x_hbm, tab_hbm, out_hbm, idx_v, rows_v, out_v, sem0, sem1):
        wid = lax.axis_index("s") * NC + lax.axis_index("c")
        pltpu.sync_copy(x_hbm.at[wid], idx_v)
        sems = (sem0, sem1)

        def start(chunk, b):
            pltpu.async_copy(tab_hbm.at[idx_v.at[chunk]], rows_v.at[b], sems[b])

        def wait(b):
            pltpu.make_async_copy(
                tab_hbm.at[idx_v.at[0]], rows_v.at[b], sems[b]
            ).wait()

        start(0, 0)
        start(1, 1)

        def accum(b, chunk):
            for kk in range(RPC):
                def body(t, acc, kk=kk):
                    base = kk * HIST + t * 5
                    for u in range(5):
                        r = base + u
                        acc = tuple(
                            acc[i] + rows_v[b, r, pl.ds(i * LANES, LANES)]
                            for i in range(NV)
                        )
                    return acc

                acc = lax.fori_loop(
                    0, HIST // 5, body,
                    tuple(jnp.zeros((LANES,), jnp.float32) for _ in range(NV)),
                )
                row = chunk * RPC + kk
                for i in range(NV):
                    out_v[row, pl.ds(i * LANES, LANES)] = acc[i]

        def outer(j, carry):
            for b in range(2):
                chunk = j * 2 + b
                wait(b)
                accum(b, chunk)

                @pl.when(chunk + 2 < NCHUNK)
                def _():
                    start(chunk + 2, b)
            return carry

        lax.fori_loop(0, NCHUNK // 2, outer, 0)
        pltpu.sync_copy(out_v, out_hbm.at[pl.ds(wid * ROWS_W, ROWS_W)])

    return k(x_r, table)


def _tc_mlp(pooled, W1, b1, W2, b2):
    BM = 512

    def body(p_ref, w1_ref, b1_ref, w2_ref, b2_ref, o_ref):
        emb = p_ref[...] * (1.0 / HIST)
        h = jnp.dot(emb, w1_ref[...], preferred_element_type=jnp.float32)
        h = jnp.maximum(h + b1_ref[...], 0.0)
        logits = jnp.dot(h, w2_ref[...], preferred_element_type=jnp.float32)
        logits = logits + b2_ref[...]
        m = jnp.max(logits, axis=1, keepdims=True)
        lse = jnp.log(jnp.sum(jnp.exp(logits - m), axis=1, keepdims=True)) + m
        o_ref[...] = logits - lse

    return pl.pallas_call(
        body,
        grid=(B // BM,),
        in_specs=[
            pl.BlockSpec((BM, D), lambda i: (i, 0)),
            pl.BlockSpec((D, H), lambda i: (0, 0)),
            pl.BlockSpec((1, H), lambda i: (0, 0)),
            pl.BlockSpec((H, C), lambda i: (0, 0)),
            pl.BlockSpec((1, C), lambda i: (0, 0)),
        ],
        out_specs=pl.BlockSpec((BM, C), lambda i: (i, 0)),
        out_shape=jax.ShapeDtypeStruct((B, C), jnp.float32),
    )(pooled, W1, b1.reshape(1, H), W2, b2.reshape(1, C))


@jax.jit
def kernel(x, table, W1, b1, W2, b2):
    x_r = x.reshape(NW, NCHUNK, IPC)
    pooled = _sc_pool(x_r, table)
    return _tc_mlp(pooled, W1, b1, W2, b2)


# speedup vs baseline: 5.8004x; 2.3204x over previous
import functools

import jax
import jax.numpy as jnp
from jax import lax
from jax.experimental import pallas as pl
from jax.experimental.pallas import tpu as pltpu
from jax.experimental.pallas import tpu_sc as plsc

B = 16384
HIST = 50
D = 64
H = 256
C = 2

NC = 2
NS = 16
NW = NC * NS
ROWS_W = B // NW
RPC = 2
IPC = RPC * HIST
NCHUNK = ROWS_W // RPC
LANES = 16
NV = D // LANES


def _sc_pool(x_r, table):
    mesh = plsc.VectorSubcoreMesh(core_axis_name="c", subcore_axis_name="s")

    @functools.partial(
        pl.kernel,
        mesh=mesh,
        compiler_params=pltpu.CompilerParams(use_tc_tiling_on_sc=False),
        out_type=jax.ShapeDtypeStruct((B, D), jnp.float32),
        scratch_types=[
            pltpu.VMEM((NCHUNK, IPC), jnp.int32),
            pltpu.VMEM((8, IPC, D), jnp.float32),
            pltpu.VMEM((ROWS_W, D), jnp.float32),
        ] + [pltpu.SemaphoreType.DMA] * 8,
    )
    def k(x_hbm, tab_hbm, out_hbm, idx_v, rows_v, out_v, *sems):
        wid = lax.axis_index("s") * NC + lax.axis_index("c")
        pltpu.sync_copy(x_hbm.at[wid], idx_v)

        def start(chunk, b):
            pltpu.async_copy(tab_hbm.at[idx_v.at[chunk]], rows_v.at[b], sems[b])

        def wait(b):
            pltpu.make_async_copy(
                tab_hbm.at[idx_v.at[0]], rows_v.at[b], sems[b]
            ).wait()

        for b in range(8):
            start(b, b)

        def accum(b, chunk):
            for kk in range(RPC):
                def body(t, acc, kk=kk):
                    base = kk * HIST + t * 5
                    for u in range(5):
                        r = base + u
                        acc = tuple(
                            acc[i] + rows_v[b, r, pl.ds(i * LANES, LANES)]
                            for i in range(NV)
                        )
                    return acc

                acc = lax.fori_loop(
                    0, HIST // 5, body,
                    tuple(jnp.zeros((LANES,), jnp.float32) for _ in range(NV)),
                )
                row = chunk * RPC + kk
                for i in range(NV):
                    out_v[row, pl.ds(i * LANES, LANES)] = acc[i]

        def outer(j, carry):
            for b in range(8):
                chunk = j * 8 + b
                wait(b)
                accum(b, chunk)

                @pl.when(chunk + 8 < NCHUNK)
                def _():
                    start(chunk + 8, b)
            return carry

        lax.fori_loop(0, NCHUNK // 8, outer, 0)
        pltpu.sync_copy(out_v, out_hbm.at[pl.ds(wid * ROWS_W, ROWS_W)])

    return k(x_r, table)


V = 1000000
HALF = 524288
V2 = 2 * HALF
VB = 16384


def _tc_repack(table):
    tT = table.T

    def body(lo_ref, hi_ref, o_ref):
        o_ref[:, 0:D] = jnp.transpose(lo_ref[...], (1, 0))
        o_ref[:, D:2 * D] = jnp.transpose(hi_ref[...], (1, 0))

    return pl.pallas_call(
        body,
        grid=(HALF // VB,),
        compiler_params=pltpu.CompilerParams(
            dimension_semantics=("parallel",),
        ),
        in_specs=[
            pl.BlockSpec((D, VB), lambda i: (0, i)),
            pl.BlockSpec((D, VB), lambda i: (0, jnp.minimum(i + HALF // VB, V // VB))),
        ],
        out_specs=pl.BlockSpec((VB, 2 * D), lambda i: (i, 0)),
        out_shape=jax.ShapeDtypeStruct((HALF, 2 * D), jnp.float32),
    )(tT, tT)


def _tc_mlp(pooled, W1, b1, W2, b2):
    BM = 2048

    def body(p_ref, w1_ref, b1_ref, w2_ref, b2_ref, o_ref):
        emb = p_ref[...] * (1.0 / HIST)
        h = jnp.dot(emb, w1_ref[...], preferred_element_type=jnp.float32)
        h = jnp.maximum(h + b1_ref[...], 0.0)
        logits = jnp.dot(h, w2_ref[...], preferred_element_type=jnp.float32)
        logits = logits + b2_ref[...]
        m = jnp.max(logits, axis=1, keepdims=True)
        lse = jnp.log(jnp.sum(jnp.exp(logits - m), axis=1, keepdims=True)) + m
        o_ref[...] = logits - lse

    return pl.pallas_call(
        body,
        grid=(B // BM,),
        in_specs=[
            pl.BlockSpec((BM, D), lambda i: (i, 0)),
            pl.BlockSpec((D, H), lambda i: (0, 0)),
            pl.BlockSpec((1, H), lambda i: (0, 0)),
            pl.BlockSpec((H, C), lambda i: (0, 0)),
            pl.BlockSpec((1, C), lambda i: (0, 0)),
        ],
        out_specs=pl.BlockSpec((BM, C), lambda i: (i, 0)),
        out_shape=jax.ShapeDtypeStruct((B, C), jnp.float32),
    )(pooled, W1, b1.reshape(1, H), W2, b2.reshape(1, C))


@jax.jit
def kernel(x, table, W1, b1, W2, b2):
    x2 = jnp.where(x < HALF, 2 * x, 2 * (x - HALF) + 1)
    x_r = x2.reshape(NW, NCHUNK, IPC)
    table_lin = _tc_repack(table).reshape(V2, D)
    pooled = _sc_pool(x_r, table_lin)
    return _tc_mlp(pooled, W1, b1, W2, b2)
